# baseline (speedup 1.0000x reference)
import jax
import jax.numpy as jnp
from jax import lax
from jax.experimental import pallas as pl
from jax.experimental.pallas import tpu as pltpu
from jax.experimental.pallas import tpu_sc as plsc

N = 10000
NP = 10240
E = 160000
D_IN = 128
H = 8
D_OUT = 32
F = H * D_OUT
FH = F // 2
HH = H // 2
HID = 128

NCORE = 2
NSUB = 16
EPT = E // NSUB
CH = 80
NCHUNK = EPT // CH
RPT = NP // NSUB

_BLK = 400
_GRID = N // _BLK


def _stage1_body(x_ref, w_ref, wb_ref, feat_ref, elr_ref):
    x = x_ref[...]
    f = jnp.dot(x, w_ref[0], preferred_element_type=jnp.float32)
    feat_ref[...] = f.reshape(1, 1, _BLK, FH)
    e = jnp.dot(x, wb_ref[0], preferred_element_type=jnp.float32)
    elr_ref[...] = e.reshape(1, _BLK, 16)


def _stage1(x, Wst, WB16):
    return pl.pallas_call(
        _stage1_body,
        grid=(2, 2, _GRID),
        in_specs=[
            pl.BlockSpec((_BLK, D_IN), lambda g, h, i: (i, 0)),
            pl.BlockSpec((1, D_IN, FH), lambda g, h, i: (g, 0, h)),
            pl.BlockSpec((1, D_IN, 16), lambda g, h, i: (g, 0, 0)),
        ],
        out_specs=[
            pl.BlockSpec((1, 1, _BLK, FH), lambda g, h, i: (g, h, i, 0)),
            pl.BlockSpec((1, _BLK, 16), lambda g, h, i: (g, i, 0)),
        ],
        out_shape=[
            jax.ShapeDtypeStruct((2, 2, N, FH), jnp.float32),
            jax.ShapeDtypeStruct((2, N, 16), jnp.float32),
        ],
    )(x, Wst, WB16)


WB = 40
NWB = RPT // WB


def _sc_body(feat, elr, sdall,
             zcat, dcat,
             sdbuf, featidx, srcelr, dstelr, dstraw,
             bufA, bufB, fbuf, exch, exflat, zbuf, dzbuf, dwbuf,
             out_sp, den_sp,
             semi0, semi1, semf0, semf1, sema0, sema1, semb0, semb1,
             semx0, semx1, semo0, semo1):
    c = lax.axis_index("c")
    s = lax.axis_index("s")
    r0 = s * RPT
    io16 = lax.iota(jnp.int32, 16)
    zv = jnp.zeros((16,), jnp.float32)
    colb = jnp.broadcast_to(HH * c, (16,))
    semi = [semi0, semi1]
    semfv = [semf0, semf1]
    semav = [sema0, sema1]
    sembv = [semb0, semb1]
    semxv = [semx0, semx1]
    semov = [semo0, semo1]

    def _zrow(i, carry):
        for j in range(8):
            zbuf[i, pl.ds(16 * j, 16)] = zv
        return carry
    lax.fori_loop(0, WB, _zrow, 0)

    def _zrow2(i, carry):
        dzbuf[i, pl.ds(0, 16)] = zv
        return carry
    lax.fori_loop(0, WB, _zrow2, 0)

    def _zrow3(i, carry):
        exch[i, pl.ds(0, 16)] = zv
        return carry
    lax.fori_loop(0, 2 * CH, _zrow3, 0)

    def _issue_idx(g, k, b):
        off = g * 2 * E + s * 2 * EPT + k * 2 * CH
        return pltpu.async_copy(
            sdall.at[pl.ds(off, 2 * CH)], sdbuf.at[b], semi[b])

    def _build_sets(g, b):
        foff = jnp.broadcast_to(g * 2 * N + c * N, (16,))
        noff = jnp.broadcast_to(g * N, (16,))
        for j in range(CH // 16):
            sv = sdbuf[b, pl.ds(16 * j, 16)]
            dv = sdbuf[b, pl.ds(CH + 16 * j, 16)]
            featidx[b, pl.ds(16 * j, 16)] = sv + foff
            srcelr[b, pl.ds(16 * j, 16)] = sv + noff
            dstelr[b, pl.ds(16 * j, 16)] = dv + noff
            dstraw[b, pl.ds(16 * j, 16)] = dv

    def _issue_gathers(b):
        bo = b * CH
        pltpu.async_copy(feat.at[featidx.at[b]],
                         fbuf.at[pl.ds(bo, CH)], semfv[b])
        pltpu.async_copy(elr.at[srcelr.at[b]],
                         bufA.at[pl.ds(bo, CH)], semav[b])
        pltpu.async_copy(elr.at[dstelr.at[b]],
                         bufB.at[pl.ds(bo, CH)], sembv[b])

    def _wait_gathers(b):
        bo = b * CH
        pltpu.make_async_copy(feat.at[featidx.at[b]],
                              fbuf.at[pl.ds(bo, CH)], semfv[b]).wait()
        pltpu.make_async_copy(elr.at[srcelr.at[b]],
                              bufA.at[pl.ds(bo, CH)], semav[b]).wait()
        pltpu.make_async_copy(elr.at[dstelr.at[b]],
                              bufB.at[pl.ds(bo, CH)], sembv[b]).wait()

    def _issue_scatters(b):
        bo = b * CH
        pltpu.async_copy(exch.at[pl.ds(bo, CH)],
                         den_sp.at[dstraw.at[b]], semxv[b], add=True)
        pltpu.async_copy(fbuf.at[pl.ds(bo, CH)],
                         out_sp.at[dstraw.at[b]], semov[b], add=True)

    def _wait_scatters(b):
        bo = b * CH
        pltpu.make_async_copy(exch.at[pl.ds(bo, CH)],
                              den_sp.at[dstraw.at[b]], semxv[b]).wait()
        pltpu.make_async_copy(fbuf.at[pl.ds(bo, CH)],
                              out_sp.at[dstraw.at[b]], semov[b]).wait()

    def _compute(b):
        bo = b * CH
        for g2 in range(CH // 16):
            rows = io16 + (16 * g2 + bo)
            rloc = io16 + 16 * g2
            for h in range(HH):
                colv = colb + h
                a = plsc.load_gather(bufA, [rows, colv])
                bb = plsc.load_gather(bufB, [rows, colv + H])
                v = a + bb
                v = jnp.where(v > 0, v, 0.2 * v)
                ex = jnp.exp(v)
                plsc.store_scatter(
                    exch, [rows, jnp.full((16,), h, jnp.int32)], ex)
                plsc.store_scatter(exflat, [rloc * HH + (h + bo * HH)], ex)

        def edge_body(q, ecarry):
            ld = exflat[pl.ds(b * CH * HH + q * 16, 16)]
            for i in range(4):
                for h in range(HH):
                    w = jnp.broadcast_to(ld[4 * i + h], (16,))
                    for j2 in range(2):
                        off = 32 * h + 16 * j2
                        fbuf[bo + q * 4 + i, pl.ds(off, 16)] = (
                            fbuf[bo + q * 4 + i, pl.ds(off, 16)] * w)
            return ecarry
        lax.fori_loop(0, CH // 4, edge_body, 0, unroll=4)

    def _chunk_step(g, k, b):
        nb = 1 - b

        @pl.when(k < NCHUNK - 1)
        def _prefetch():
            pltpu.make_async_copy(
                sdall.at[pl.ds(0, 2 * CH)], sdbuf.at[nb], semi[nb]).wait()

            @pl.when(k >= 1)
            def _drain_prev():
                _wait_scatters(nb)
            _build_sets(g, nb)
            _issue_gathers(nb)

        @pl.when(k < NCHUNK - 2)
        def _next_idx():
            _issue_idx(g, k + 2, b)

        _wait_gathers(b)
        _compute(b)
        _issue_scatters(b)

    def graph_body(g, gcarry):
        ooff = g * 2 * NP + c * NP + r0

        for i in range(NWB):
            pltpu.sync_copy(zbuf, out_sp.at[pl.ds(r0 + WB * i, WB)])
            pltpu.sync_copy(dzbuf, den_sp.at[pl.ds(r0 + WB * i, WB)])
        plsc.subcore_barrier()

        _issue_idx(g, 0, 0).wait()
        _build_sets(g, 0)
        _issue_gathers(0)
        _issue_idx(g, 1, 1)

        def chunk_pair(i, carry):
            _chunk_step(g, 2 * i, 0)
            _chunk_step(g, 2 * i + 1, 1)
            return carry
        lax.fori_loop(0, NCHUNK // 2, chunk_pair, 0)
        _chunk_step(g, jnp.int32(NCHUNK - 1), (NCHUNK - 1) % 2)
        _wait_scatters((NCHUNK - 1) % 2)
        _wait_scatters(NCHUNK % 2)
        plsc.subcore_barrier()

        for i in range(NWB):
            pltpu.sync_copy(out_sp.at[pl.ds(r0 + WB * i, WB)],
                            fbuf.at[pl.ds(0, WB)])
            pltpu.sync_copy(fbuf.at[pl.ds(0, WB)],
                            zcat.at[pl.ds(ooff + WB * i, WB)])
            pltpu.sync_copy(den_sp.at[pl.ds(r0 + WB * i, WB)], dwbuf)
            pltpu.sync_copy(dwbuf, dcat.at[pl.ds(ooff + WB * i, WB)])
        plsc.subcore_barrier()
        return gcarry

    lax.fori_loop(0, 2, graph_body, 0)


def _stage2(featall, elrall, sdall):
    mesh = plsc.VectorSubcoreMesh(core_axis_name="c", subcore_axis_name="s")
    f32 = jnp.float32
    i32 = jnp.int32
    return pl.kernel(
        _sc_body,
        out_type=[
            jax.ShapeDtypeStruct((4 * NP, FH), f32),
            jax.ShapeDtypeStruct((4 * NP, 16), f32),
        ],
        mesh=mesh,
        compiler_params=pltpu.CompilerParams(
            use_tc_tiling_on_sc=False, needs_layout_passes=False),
        scratch_types=[
            pltpu.VMEM((2, 2 * CH), i32),
            pltpu.VMEM((2, CH), i32),
            pltpu.VMEM((2, CH), i32),
            pltpu.VMEM((2, CH), i32),
            pltpu.VMEM((2, CH), i32),
            pltpu.VMEM((2 * CH, 16), f32),
            pltpu.VMEM((2 * CH, 16), f32),
            pltpu.VMEM((2 * CH, FH), f32),
            pltpu.VMEM((2 * CH, 16), f32),
            pltpu.VMEM((2 * CH * HH,), f32),
            pltpu.VMEM((WB, FH), f32),
            pltpu.VMEM((WB, 16), f32),
            pltpu.VMEM((WB, 16), f32),
            pltpu.VMEM_SHARED((NP, FH), f32),
            pltpu.VMEM_SHARED((NP, 16), f32),
        ] + [pltpu.SemaphoreType.DMA] * 12,
    )(featall, elrall, sdall)


_BLK3 = 80
_GRID3 = N // _BLK3
_OFF3 = NP // _BLK3


def _stage3_body(r0lo, r0hi, r1lo, r1hi, d0lo, d0hi, d1lo, d1hi,
                 b0_ref, b1_ref, slo_ref, shi_ref,
                 w1_ref, sb1_ref, w2_ref, out_ref):
    slo = slo_ref[...]
    shi = shi_ref[...]

    def z(rlo, rhi, dlo, dhi, b_ref):
        dexp = (jnp.dot(1.0 / (dlo[...] + 1e-9), slo,
                        preferred_element_type=jnp.float32)
                + jnp.dot(1.0 / (dhi[...] + 1e-9), shi,
                          preferred_element_type=jnp.float32))
        raw = jnp.concatenate([rlo[...], rhi[...]], axis=1)
        zz = raw * dexp + b_ref[...]
        return jnp.where(zz > 0, zz, jnp.exp(jnp.minimum(zz, 0.0)) - 1.0)

    z0 = z(r0lo, r0hi, d0lo, d0hi, b0_ref)
    z1 = z(r1lo, r1hi, d1lo, d1hi, b1_ref)
    b1v = sb1_ref[...]
    w2 = w2_ref[...]
    h0 = jnp.tanh(jnp.dot(z0, w1_ref[...], preferred_element_type=jnp.float32) + b1v)
    h1 = jnp.tanh(jnp.dot(z1, w1_ref[...], preferred_element_type=jnp.float32) + b1v)
    s0 = jnp.sum(h0 * w2, axis=1, keepdims=True)
    s1 = jnp.sum(h1 * w2, axis=1, keepdims=True)
    beta0 = jax.nn.sigmoid(s0 - s1)
    out_ref[...] = z1 + beta0 * (z0 - z1)


def _stage3(zcat, dcat, bias_0, bias_1, SLO, SHI, sem_W1, sem_b1, sem_W2):
    zspec = [pl.BlockSpec((_BLK3, FH), lambda i, o=o: (o * _OFF3 + i, 0))
             for o in range(4)]
    dspec = [pl.BlockSpec((_BLK3, 16), lambda i, o=o: (o * _OFF3 + i, 0))
             for o in range(4)]
    return pl.pallas_call(
        _stage3_body,
        grid=(_GRID3,),
        in_specs=zspec + dspec + [
            pl.BlockSpec((1, F), lambda i: (0, 0)),
            pl.BlockSpec((1, F), lambda i: (0, 0)),
            pl.BlockSpec((16, F), lambda i: (0, 0)),
            pl.BlockSpec((16, F), lambda i: (0, 0)),
            pl.BlockSpec((F, HID), lambda i: (0, 0)),
            pl.BlockSpec((1, HID), lambda i: (0, 0)),
            pl.BlockSpec((1, HID), lambda i: (0, 0)),
        ],
        out_specs=pl.BlockSpec((_BLK3, F), lambda i: (i, 0)),
        out_shape=jax.ShapeDtypeStruct((N, F), jnp.float32),
    )(zcat, zcat, zcat, zcat, dcat, dcat, dcat, dcat,
      bias_0.reshape(1, F), bias_1.reshape(1, F), SLO, SHI,
      sem_W1, sem_b1.reshape(1, HID), sem_W2.reshape(1, HID))


def _attn_mat(attn):
    mask = jnp.repeat(jnp.eye(H, dtype=jnp.float32), D_OUT, axis=0)
    return mask * jnp.tile(attn.T, (H, 1))


def kernel(x, edge_index_0, edge_index_1, W_0, attn_l_0, attn_r_0, bias_0,
           W_1, attn_l_1, attn_r_1, bias_1, sem_W1, sem_b1, sem_W2, sem_b2):
    del sem_b2
    B_0 = jnp.concatenate([_attn_mat(attn_l_0), _attn_mat(attn_r_0)], axis=1)
    B_1 = jnp.concatenate([_attn_mat(attn_l_1), _attn_mat(attn_r_1)], axis=1)
    Wst = jnp.stack([W_0, W_1], axis=0)
    WB16 = jnp.stack([W_0 @ B_0, W_1 @ B_1], axis=0)
    featall4, elrall3 = _stage1(x, Wst, WB16)
    featall = featall4.reshape(4 * N, FH)
    elrall = elrall3.reshape(2 * N, 16)
    srcall = jnp.concatenate([edge_index_0[0], edge_index_1[0]], axis=0)
    dstall = jnp.concatenate([edge_index_0[1], edge_index_1[1]], axis=0)
    sdall = jnp.stack(
        [srcall.reshape(2, NSUB, NCHUNK, CH), dstall.reshape(2, NSUB, NCHUNK, CH)],
        axis=3).reshape(-1)
    zcat, dcat = _stage2(featall, elrall, sdall)
    base = jnp.repeat(jnp.eye(H, dtype=jnp.float32), D_OUT, axis=1)
    zpad = jnp.zeros((12, F), jnp.float32)
    SLO = jnp.concatenate([base[:HH], zpad], axis=0)
    SHI = jnp.concatenate([base[HH:], zpad], axis=0)
    return _stage3(zcat, dcat, bias_0, bias_1, SLO, SHI,
                   sem_W1, sem_b1, sem_W2)


# speedup vs baseline: 1.1927x; 1.1927x over previous
import jax
import jax.numpy as jnp
from jax import lax
from jax.experimental import pallas as pl
from jax.experimental.pallas import tpu as pltpu
from jax.experimental.pallas import tpu_sc as plsc

N = 10000
NP = 10240
E = 160000
D_IN = 128
H = 8
D_OUT = 32
F = H * D_OUT
FH = F // 2
HH = H // 2
HID = 128

NCORE = 2
NSUB = 16
EPT = E // NSUB
CH = 80
NCHUNK = EPT // CH
RPT = NP // NSUB

_BLK = 2000
_GRID = N // _BLK


def _stage1_body(x_ref, w_ref, wb_ref, feat_ref, elr_ref):
    x = x_ref[...]
    f = jnp.dot(x, w_ref[0], preferred_element_type=jnp.float32)
    feat_ref[...] = f.reshape(1, 1, _BLK, FH)
    e = jnp.dot(x, wb_ref[0], preferred_element_type=jnp.float32)
    elr_ref[...] = e.reshape(1, _BLK, 16)


def _stage1(x, Wst, WB16):
    return pl.pallas_call(
        _stage1_body,
        grid=(2, 2, _GRID),
        in_specs=[
            pl.BlockSpec((_BLK, D_IN), lambda g, h, i: (i, 0)),
            pl.BlockSpec((1, D_IN, FH), lambda g, h, i: (g, 0, h)),
            pl.BlockSpec((1, D_IN, 16), lambda g, h, i: (g, 0, 0)),
        ],
        out_specs=[
            pl.BlockSpec((1, 1, _BLK, FH), lambda g, h, i: (g, h, i, 0)),
            pl.BlockSpec((1, _BLK, 16), lambda g, h, i: (g, i, 0)),
        ],
        out_shape=[
            jax.ShapeDtypeStruct((2, 2, N, FH), jnp.float32),
            jax.ShapeDtypeStruct((2, N, 16), jnp.float32),
        ],
    )(x, Wst, WB16)


WB = 40
NWB = RPT // WB


def _sc_body(feat, elr, sdall,
             zcat, dcat,
             sdbuf, featidx, srcelr, dstelr, dstraw,
             bufA, bufB, fbuf, exch, exflat, zbuf, dzbuf, dwbuf,
             out_sp, den_sp,
             semi0, semi1, semf0, semf1, sema0, sema1, semb0, semb1,
             semx0, semx1, semo0, semo1):
    c = lax.axis_index("c")
    s = lax.axis_index("s")
    r0 = s * RPT
    io16 = lax.iota(jnp.int32, 16)
    zv = jnp.zeros((16,), jnp.float32)
    colb = jnp.broadcast_to(HH * c, (16,))
    semi = [semi0, semi1]
    semfv = [semf0, semf1]
    semav = [sema0, sema1]
    sembv = [semb0, semb1]
    semxv = [semx0, semx1]
    semov = [semo0, semo1]

    def _zrow(i, carry):
        for j in range(8):
            zbuf[i, pl.ds(16 * j, 16)] = zv
        return carry
    lax.fori_loop(0, WB, _zrow, 0)

    def _zrow2(i, carry):
        dzbuf[i, pl.ds(0, 16)] = zv
        return carry
    lax.fori_loop(0, WB, _zrow2, 0)

    def _zrow3(i, carry):
        exch[i, pl.ds(0, 16)] = zv
        return carry
    lax.fori_loop(0, 2 * CH, _zrow3, 0)

    def _issue_idx(g, k, b):
        off = g * 2 * E + s * 2 * EPT + k * 2 * CH
        return pltpu.async_copy(
            sdall.at[pl.ds(off, 2 * CH)], sdbuf.at[b], semi[b])

    def _build_sets(g, b):
        foff = jnp.broadcast_to(g * 2 * N + c * N, (16,))
        noff = jnp.broadcast_to(g * N, (16,))
        for j in range(CH // 16):
            sv = sdbuf[b, pl.ds(16 * j, 16)]
            dv = sdbuf[b, pl.ds(CH + 16 * j, 16)]
            featidx[b, pl.ds(16 * j, 16)] = sv + foff
            srcelr[b, pl.ds(16 * j, 16)] = sv + noff
            dstelr[b, pl.ds(16 * j, 16)] = dv + noff
            dstraw[b, pl.ds(16 * j, 16)] = dv

    def _issue_gathers(b):
        bo = b * CH
        pltpu.async_copy(feat.at[featidx.at[b]],
                         fbuf.at[pl.ds(bo, CH)], semfv[b])
        pltpu.async_copy(elr.at[srcelr.at[b]],
                         bufA.at[pl.ds(bo, CH)], semav[b])
        pltpu.async_copy(elr.at[dstelr.at[b]],
                         bufB.at[pl.ds(bo, CH)], sembv[b])

    def _wait_gathers(b):
        bo = b * CH
        pltpu.make_async_copy(feat.at[featidx.at[b]],
                              fbuf.at[pl.ds(bo, CH)], semfv[b]).wait()
        pltpu.make_async_copy(elr.at[srcelr.at[b]],
                              bufA.at[pl.ds(bo, CH)], semav[b]).wait()
        pltpu.make_async_copy(elr.at[dstelr.at[b]],
                              bufB.at[pl.ds(bo, CH)], sembv[b]).wait()

    def _issue_scatters(b):
        bo = b * CH
        pltpu.async_copy(exch.at[pl.ds(bo, CH)],
                         den_sp.at[dstraw.at[b]], semxv[b], add=True)
        pltpu.async_copy(fbuf.at[pl.ds(bo, CH)],
                         out_sp.at[dstraw.at[b]], semov[b], add=True)

    def _wait_scatters(b):
        bo = b * CH
        pltpu.make_async_copy(exch.at[pl.ds(bo, CH)],
                              den_sp.at[dstraw.at[b]], semxv[b]).wait()
        pltpu.make_async_copy(fbuf.at[pl.ds(bo, CH)],
                              out_sp.at[dstraw.at[b]], semov[b]).wait()

    def _compute(b):
        bo = b * CH
        for g2 in range(CH // 16):
            rows = io16 + (16 * g2 + bo)
            rloc = io16 + 16 * g2
            for h in range(HH):
                colv = colb + h
                a = plsc.load_gather(bufA, [rows, colv])
                bb = plsc.load_gather(bufB, [rows, colv + H])
                v = a + bb
                v = jnp.where(v > 0, v, 0.2 * v)
                ex = jnp.exp(v)
                plsc.store_scatter(
                    exch, [rows, jnp.full((16,), h, jnp.int32)], ex)
                plsc.store_scatter(exflat, [rloc * HH + (h + bo * HH)], ex)

        def edge_body(q, ecarry):
            ld = exflat[pl.ds(b * CH * HH + q * 16, 16)]
            for i in range(4):
                for h in range(HH):
                    w = jnp.broadcast_to(ld[4 * i + h], (16,))
                    for j2 in range(2):
                        off = 32 * h + 16 * j2
                        fbuf[bo + q * 4 + i, pl.ds(off, 16)] = (
                            fbuf[bo + q * 4 + i, pl.ds(off, 16)] * w)
            return ecarry
        lax.fori_loop(0, CH // 4, edge_body, 0, unroll=4)

    def _chunk_step(g, k, b):
        nb = 1 - b

        @pl.when(k < NCHUNK - 1)
        def _prefetch():
            pltpu.make_async_copy(
                sdall.at[pl.ds(0, 2 * CH)], sdbuf.at[nb], semi[nb]).wait()

            @pl.when(k >= 1)
            def _drain_prev():
                _wait_scatters(nb)
            _build_sets(g, nb)
            _issue_gathers(nb)

        @pl.when(k < NCHUNK - 2)
        def _next_idx():
            _issue_idx(g, k + 2, b)

        _wait_gathers(b)
        _compute(b)
        _issue_scatters(b)

    def graph_body(g, gcarry):
        ooff = g * 2 * NP + c * NP + r0

        for i in range(NWB):
            pltpu.sync_copy(zbuf, out_sp.at[pl.ds(r0 + WB * i, WB)])
            pltpu.sync_copy(dzbuf, den_sp.at[pl.ds(r0 + WB * i, WB)])
        plsc.subcore_barrier()

        _issue_idx(g, 0, 0).wait()
        _build_sets(g, 0)
        _issue_gathers(0)
        _issue_idx(g, 1, 1)

        def chunk_pair(i, carry):
            _chunk_step(g, 2 * i, 0)
            _chunk_step(g, 2 * i + 1, 1)
            return carry
        lax.fori_loop(0, NCHUNK // 2, chunk_pair, 0)
        _chunk_step(g, jnp.int32(NCHUNK - 1), (NCHUNK - 1) % 2)
        _wait_scatters((NCHUNK - 1) % 2)
        _wait_scatters(NCHUNK % 2)
        plsc.subcore_barrier()

        for i in range(NWB):
            pltpu.sync_copy(out_sp.at[pl.ds(r0 + WB * i, WB)],
                            fbuf.at[pl.ds(0, WB)])
            pltpu.sync_copy(fbuf.at[pl.ds(0, WB)],
                            zcat.at[pl.ds(ooff + WB * i, WB)])
            pltpu.sync_copy(den_sp.at[pl.ds(r0 + WB * i, WB)], dwbuf)
            pltpu.sync_copy(dwbuf, dcat.at[pl.ds(ooff + WB * i, WB)])
        plsc.subcore_barrier()
        return gcarry

    lax.fori_loop(0, 2, graph_body, 0)


def _stage2(featall, elrall, sdall):
    mesh = plsc.VectorSubcoreMesh(core_axis_name="c", subcore_axis_name="s")
    f32 = jnp.float32
    i32 = jnp.int32
    return pl.kernel(
        _sc_body,
        out_type=[
            jax.ShapeDtypeStruct((4 * NP, FH), f32),
            jax.ShapeDtypeStruct((4 * NP, 16), f32),
        ],
        mesh=mesh,
        compiler_params=pltpu.CompilerParams(
            use_tc_tiling_on_sc=False, needs_layout_passes=False),
        scratch_types=[
            pltpu.VMEM((2, 2 * CH), i32),
            pltpu.VMEM((2, CH), i32),
            pltpu.VMEM((2, CH), i32),
            pltpu.VMEM((2, CH), i32),
            pltpu.VMEM((2, CH), i32),
            pltpu.VMEM((2 * CH, 16), f32),
            pltpu.VMEM((2 * CH, 16), f32),
            pltpu.VMEM((2 * CH, FH), f32),
            pltpu.VMEM((2 * CH, 16), f32),
            pltpu.VMEM((2 * CH * HH,), f32),
            pltpu.VMEM((WB, FH), f32),
            pltpu.VMEM((WB, 16), f32),
            pltpu.VMEM((WB, 16), f32),
            pltpu.VMEM_SHARED((NP, FH), f32),
            pltpu.VMEM_SHARED((NP, 16), f32),
        ] + [pltpu.SemaphoreType.DMA] * 12,
    )(featall, elrall, sdall)


_BLK3 = 640
_GRID3 = NP // _BLK3
_OFF3 = NP // _BLK3


def _stage3_body(r0lo, r0hi, r1lo, r1hi, d0lo, d0hi, d1lo, d1hi,
                 b0_ref, b1_ref, slo_ref, shi_ref,
                 w1_ref, sb1_ref, w2_ref, out_ref):
    slo = slo_ref[...]
    shi = shi_ref[...]

    def z(rlo, rhi, dlo, dhi, b_ref):
        dexp = (jnp.dot(1.0 / (dlo[...] + 1e-9), slo,
                        preferred_element_type=jnp.float32)
                + jnp.dot(1.0 / (dhi[...] + 1e-9), shi,
                          preferred_element_type=jnp.float32))
        raw = jnp.concatenate([rlo[...], rhi[...]], axis=1)
        zz = raw * dexp + b_ref[...]
        return jnp.where(zz > 0, zz, jnp.exp(jnp.minimum(zz, 0.0)) - 1.0)

    z0 = z(r0lo, r0hi, d0lo, d0hi, b0_ref)
    z1 = z(r1lo, r1hi, d1lo, d1hi, b1_ref)
    b1v = sb1_ref[...]
    w2 = w2_ref[...]
    h0 = jnp.tanh(jnp.dot(z0, w1_ref[...], preferred_element_type=jnp.float32) + b1v)
    h1 = jnp.tanh(jnp.dot(z1, w1_ref[...], preferred_element_type=jnp.float32) + b1v)
    s0 = jnp.sum(h0 * w2, axis=1, keepdims=True)
    s1 = jnp.sum(h1 * w2, axis=1, keepdims=True)
    beta0 = jax.nn.sigmoid(s0 - s1)
    out_ref[...] = z1 + beta0 * (z0 - z1)


def _stage3(zcat, dcat, bias_0, bias_1, SLO, SHI, sem_W1, sem_b1, sem_W2):
    zspec = [pl.BlockSpec((_BLK3, FH), lambda i, o=o: (o * _OFF3 + i, 0))
             for o in range(4)]
    dspec = [pl.BlockSpec((_BLK3, 16), lambda i, o=o: (o * _OFF3 + i, 0))
             for o in range(4)]
    return pl.pallas_call(
        _stage3_body,
        grid=(_GRID3,),
        in_specs=zspec + dspec + [
            pl.BlockSpec((1, F), lambda i: (0, 0)),
            pl.BlockSpec((1, F), lambda i: (0, 0)),
            pl.BlockSpec((16, F), lambda i: (0, 0)),
            pl.BlockSpec((16, F), lambda i: (0, 0)),
            pl.BlockSpec((F, HID), lambda i: (0, 0)),
            pl.BlockSpec((1, HID), lambda i: (0, 0)),
            pl.BlockSpec((1, HID), lambda i: (0, 0)),
        ],
        out_specs=pl.BlockSpec((_BLK3, F), lambda i: (i, 0)),
        out_shape=jax.ShapeDtypeStruct((NP, F), jnp.float32),
    )(zcat, zcat, zcat, zcat, dcat, dcat, dcat, dcat,
      bias_0.reshape(1, F), bias_1.reshape(1, F), SLO, SHI,
      sem_W1, sem_b1.reshape(1, HID), sem_W2.reshape(1, HID))


def _attn_mat(attn):
    mask = jnp.repeat(jnp.eye(H, dtype=jnp.float32), D_OUT, axis=0)
    return mask * jnp.tile(attn.T, (H, 1))


def kernel(x, edge_index_0, edge_index_1, W_0, attn_l_0, attn_r_0, bias_0,
           W_1, attn_l_1, attn_r_1, bias_1, sem_W1, sem_b1, sem_W2, sem_b2):
    del sem_b2
    B_0 = jnp.concatenate([_attn_mat(attn_l_0), _attn_mat(attn_r_0)], axis=1)
    B_1 = jnp.concatenate([_attn_mat(attn_l_1), _attn_mat(attn_r_1)], axis=1)
    Wst = jnp.stack([W_0, W_1], axis=0)
    WB16 = jnp.stack([W_0 @ B_0, W_1 @ B_1], axis=0)
    featall4, elrall3 = _stage1(x, Wst, WB16)
    featall = featall4.reshape(4 * N, FH)
    elrall = elrall3.reshape(2 * N, 16)
    srcall = jnp.concatenate([edge_index_0[0], edge_index_1[0]], axis=0)
    dstall = jnp.concatenate([edge_index_0[1], edge_index_1[1]], axis=0)
    sdall = jnp.stack(
        [srcall.reshape(2, NSUB, NCHUNK, CH), dstall.reshape(2, NSUB, NCHUNK, CH)],
        axis=3).reshape(-1)
    zcat, dcat = _stage2(featall, elrall, sdall)
    base = jnp.repeat(jnp.eye(H, dtype=jnp.float32), D_OUT, axis=1)
    zpad = jnp.zeros((12, F), jnp.float32)
    SLO = jnp.concatenate([base[:HH], zpad], axis=0)
    SHI = jnp.concatenate([base[HH:], zpad], axis=0)
    return _stage3(zcat, dcat, bias_0, bias_1, SLO, SHI,
                   sem_W1, sem_b1, sem_W2)[:N]


# speedup vs baseline: 1.2312x; 1.0323x over previous
import jax
import jax.numpy as jnp
from jax import lax
from jax.experimental import pallas as pl
from jax.experimental.pallas import tpu as pltpu
from jax.experimental.pallas import tpu_sc as plsc

N = 10000
NP = 10240
E = 160000
D_IN = 128
H = 8
D_OUT = 32
F = H * D_OUT
FH = F // 2
HH = H // 2
HID = 128

NCORE = 2
NSUB = 16
EPT = E // NSUB
CH = 80
NCHUNK = EPT // CH
RPT = NP // NSUB

_BLK = 10000
_GRID = N // _BLK


def _stage1_body(x_ref, w_ref, wb_ref, feat_ref, elr_ref):
    x = x_ref[...]
    f = jnp.dot(x, w_ref[0], preferred_element_type=jnp.float32)
    feat_ref[...] = f.reshape(1, 1, _BLK, FH)
    e = jnp.dot(x, wb_ref[0], preferred_element_type=jnp.float32)
    elr_ref[...] = e.reshape(1, _BLK, 16)


def _stage1(x, Wst, WB16):
    return pl.pallas_call(
        _stage1_body,
        grid=(2, 2, _GRID),
        in_specs=[
            pl.BlockSpec((_BLK, D_IN), lambda g, h, i: (i, 0)),
            pl.BlockSpec((1, D_IN, FH), lambda g, h, i: (g, 0, h)),
            pl.BlockSpec((1, D_IN, 16), lambda g, h, i: (g, 0, 0)),
        ],
        out_specs=[
            pl.BlockSpec((1, 1, _BLK, FH), lambda g, h, i: (g, h, i, 0)),
            pl.BlockSpec((1, _BLK, 16), lambda g, h, i: (g, i, 0)),
        ],
        out_shape=[
            jax.ShapeDtypeStruct((2, 2, N, FH), jnp.float32),
            jax.ShapeDtypeStruct((2, N, 16), jnp.float32),
        ],
    )(x, Wst, WB16)


WB = 40
NWB = RPT // WB


def _sc_body(feat, elr, sdall,
             zcat, dcat,
             sdbuf, featidx, srcelr, dstelr, dstraw,
             bufA, bufB, fbuf, exch, exflat, zbuf, dzbuf, dwbuf,
             out_sp, den_sp,
             semi0, semi1, semf0, semf1, sema0, sema1, semb0, semb1,
             semx0, semx1, semo0, semo1):
    c = lax.axis_index("c")
    s = lax.axis_index("s")
    r0 = s * RPT
    io16 = lax.iota(jnp.int32, 16)
    zv = jnp.zeros((16,), jnp.float32)
    colb = jnp.broadcast_to(HH * c, (16,))
    semi = [semi0, semi1]
    semfv = [semf0, semf1]
    semav = [sema0, sema1]
    sembv = [semb0, semb1]
    semxv = [semx0, semx1]
    semov = [semo0, semo1]

    def _zrow(i, carry):
        for j in range(8):
            zbuf[i, pl.ds(16 * j, 16)] = zv
        return carry
    lax.fori_loop(0, WB, _zrow, 0)

    def _zrow2(i, carry):
        dzbuf[i, pl.ds(0, 16)] = zv
        return carry
    lax.fori_loop(0, WB, _zrow2, 0)

    def _zrow3(i, carry):
        exch[i, pl.ds(0, 16)] = zv
        return carry
    lax.fori_loop(0, 2 * CH, _zrow3, 0)

    def _issue_idx(g, k, b):
        off = g * 2 * E + s * 2 * EPT + k * 2 * CH
        return pltpu.async_copy(
            sdall.at[pl.ds(off, 2 * CH)], sdbuf.at[b], semi[b])

    def _build_sets(g, b):
        foff = jnp.broadcast_to(g * 2 * N + c * N, (16,))
        noff = jnp.broadcast_to(g * N, (16,))
        for j in range(CH // 16):
            sv = sdbuf[b, pl.ds(16 * j, 16)]
            dv = sdbuf[b, pl.ds(CH + 16 * j, 16)]
            featidx[b, pl.ds(16 * j, 16)] = sv + foff
            srcelr[b, pl.ds(16 * j, 16)] = sv + noff
            dstelr[b, pl.ds(16 * j, 16)] = dv + noff
            dstraw[b, pl.ds(16 * j, 16)] = dv

    def _issue_gathers(b):
        bo = b * CH
        pltpu.async_copy(feat.at[featidx.at[b]],
                         fbuf.at[pl.ds(bo, CH)], semfv[b])
        pltpu.async_copy(elr.at[srcelr.at[b]],
                         bufA.at[pl.ds(bo, CH)], semav[b])
        pltpu.async_copy(elr.at[dstelr.at[b]],
                         bufB.at[pl.ds(bo, CH)], sembv[b])

    def _wait_gathers(b):
        bo = b * CH
        pltpu.make_async_copy(feat.at[featidx.at[b]],
                              fbuf.at[pl.ds(bo, CH)], semfv[b]).wait()
        pltpu.make_async_copy(elr.at[srcelr.at[b]],
                              bufA.at[pl.ds(bo, CH)], semav[b]).wait()
        pltpu.make_async_copy(elr.at[dstelr.at[b]],
                              bufB.at[pl.ds(bo, CH)], sembv[b]).wait()

    def _issue_scatters(b):
        bo = b * CH
        pltpu.async_copy(exch.at[pl.ds(bo, CH)],
                         den_sp.at[dstraw.at[b]], semxv[b], add=True)
        pltpu.async_copy(fbuf.at[pl.ds(bo, CH)],
                         out_sp.at[dstraw.at[b]], semov[b], add=True)

    def _wait_scatters(b):
        bo = b * CH
        pltpu.make_async_copy(exch.at[pl.ds(bo, CH)],
                              den_sp.at[dstraw.at[b]], semxv[b]).wait()
        pltpu.make_async_copy(fbuf.at[pl.ds(bo, CH)],
                              out_sp.at[dstraw.at[b]], semov[b]).wait()

    def _compute(b):
        bo = b * CH
        for g2 in range(CH // 16):
            rows = io16 + (16 * g2 + bo)
            rloc = io16 + 16 * g2
            for h in range(HH):
                colv = colb + h
                a = plsc.load_gather(bufA, [rows, colv])
                bb = plsc.load_gather(bufB, [rows, colv + H])
                v = a + bb
                v = jnp.where(v > 0, v, 0.2 * v)
                ex = jnp.exp(v)
                plsc.store_scatter(
                    exch, [rows, jnp.full((16,), h, jnp.int32)], ex)
                plsc.store_scatter(exflat, [rloc * HH + (h + bo * HH)], ex)

        def edge_body(q, ecarry):
            ld = exflat[pl.ds(b * CH * HH + q * 16, 16)]
            for i in range(4):
                for h in range(HH):
                    w = jnp.broadcast_to(ld[4 * i + h], (16,))
                    for j2 in range(2):
                        off = 32 * h + 16 * j2
                        fbuf[bo + q * 4 + i, pl.ds(off, 16)] = (
                            fbuf[bo + q * 4 + i, pl.ds(off, 16)] * w)
            return ecarry
        lax.fori_loop(0, CH // 4, edge_body, 0, unroll=4)

    def _chunk_step(g, k, b):
        nb = 1 - b

        @pl.when(k < NCHUNK - 1)
        def _prefetch():
            pltpu.make_async_copy(
                sdall.at[pl.ds(0, 2 * CH)], sdbuf.at[nb], semi[nb]).wait()

            @pl.when(k >= 1)
            def _drain_prev():
                _wait_scatters(nb)
            _build_sets(g, nb)
            _issue_gathers(nb)

        @pl.when(k < NCHUNK - 2)
        def _next_idx():
            _issue_idx(g, k + 2, b)

        _wait_gathers(b)
        _compute(b)
        _issue_scatters(b)

    def graph_body(g, gcarry):
        ooff = g * 2 * NP + c * NP + r0

        for i in range(NWB):
            pltpu.sync_copy(zbuf, out_sp.at[pl.ds(r0 + WB * i, WB)])
            pltpu.sync_copy(dzbuf, den_sp.at[pl.ds(r0 + WB * i, WB)])
        plsc.subcore_barrier()

        _issue_idx(g, 0, 0).wait()
        _build_sets(g, 0)
        _issue_gathers(0)
        _issue_idx(g, 1, 1)

        def chunk_pair(i, carry):
            _chunk_step(g, 2 * i, 0)
            _chunk_step(g, 2 * i + 1, 1)
            return carry
        lax.fori_loop(0, NCHUNK // 2, chunk_pair, 0)
        _chunk_step(g, jnp.int32(NCHUNK - 1), (NCHUNK - 1) % 2)
        _wait_scatters((NCHUNK - 1) % 2)
        _wait_scatters(NCHUNK % 2)
        plsc.subcore_barrier()

        for i in range(NWB):
            pltpu.sync_copy(out_sp.at[pl.ds(r0 + WB * i, WB)],
                            fbuf.at[pl.ds(0, WB)])
            pltpu.sync_copy(fbuf.at[pl.ds(0, WB)],
                            zcat.at[pl.ds(ooff + WB * i, WB)])
            pltpu.sync_copy(den_sp.at[pl.ds(r0 + WB * i, WB)], dwbuf)
            pltpu.sync_copy(dwbuf, dcat.at[pl.ds(ooff + WB * i, WB)])
        plsc.subcore_barrier()
        return gcarry

    lax.fori_loop(0, 2, graph_body, 0)


def _stage2(featall, elrall, sdall):
    mesh = plsc.VectorSubcoreMesh(core_axis_name="c", subcore_axis_name="s")
    f32 = jnp.float32
    i32 = jnp.int32
    return pl.kernel(
        _sc_body,
        out_type=[
            jax.ShapeDtypeStruct((4 * NP, FH), f32),
            jax.ShapeDtypeStruct((4 * NP, 16), f32),
        ],
        mesh=mesh,
        compiler_params=pltpu.CompilerParams(
            use_tc_tiling_on_sc=False, needs_layout_passes=False),
        scratch_types=[
            pltpu.VMEM((2, 2 * CH), i32),
            pltpu.VMEM((2, CH), i32),
            pltpu.VMEM((2, CH), i32),
            pltpu.VMEM((2, CH), i32),
            pltpu.VMEM((2, CH), i32),
            pltpu.VMEM((2 * CH, 16), f32),
            pltpu.VMEM((2 * CH, 16), f32),
            pltpu.VMEM((2 * CH, FH), f32),
            pltpu.VMEM((2 * CH, 16), f32),
            pltpu.VMEM((2 * CH * HH,), f32),
            pltpu.VMEM((WB, FH), f32),
            pltpu.VMEM((WB, 16), f32),
            pltpu.VMEM((WB, 16), f32),
            pltpu.VMEM_SHARED((NP, FH), f32),
            pltpu.VMEM_SHARED((NP, 16), f32),
        ] + [pltpu.SemaphoreType.DMA] * 12,
    )(featall, elrall, sdall)


_BLK3 = 1280
_GRID3 = NP // _BLK3
_OFF3 = NP // _BLK3


def _stage3_body(r0lo, r0hi, r1lo, r1hi, d0lo, d0hi, d1lo, d1hi,
                 b0_ref, b1_ref, slo_ref, shi_ref,
                 w1_ref, sb1_ref, w2_ref, out_ref):
    slo = slo_ref[...]
    shi = shi_ref[...]

    def z(rlo, rhi, dlo, dhi, b_ref):
        dexp = (jnp.dot(1.0 / (dlo[...] + 1e-9), slo,
                        preferred_element_type=jnp.float32)
                + jnp.dot(1.0 / (dhi[...] + 1e-9), shi,
                          preferred_element_type=jnp.float32))
        raw = jnp.concatenate([rlo[...], rhi[...]], axis=1)
        zz = raw * dexp + b_ref[...]
        return jnp.where(zz > 0, zz, jnp.exp(jnp.minimum(zz, 0.0)) - 1.0)

    z0 = z(r0lo, r0hi, d0lo, d0hi, b0_ref)
    z1 = z(r1lo, r1hi, d1lo, d1hi, b1_ref)
    b1v = sb1_ref[...]
    w2 = w2_ref[...]
    h0 = jnp.tanh(jnp.dot(z0, w1_ref[...], preferred_element_type=jnp.float32) + b1v)
    h1 = jnp.tanh(jnp.dot(z1, w1_ref[...], preferred_element_type=jnp.float32) + b1v)
    s0 = jnp.sum(h0 * w2, axis=1, keepdims=True)
    s1 = jnp.sum(h1 * w2, axis=1, keepdims=True)
    beta0 = jax.nn.sigmoid(s0 - s1)
    out_ref[...] = z1 + beta0 * (z0 - z1)


def _stage3(zcat, dcat, bias_0, bias_1, SLO, SHI, sem_W1, sem_b1, sem_W2):
    zspec = [pl.BlockSpec((_BLK3, FH), lambda i, o=o: (o * _OFF3 + i, 0))
             for o in range(4)]
    dspec = [pl.BlockSpec((_BLK3, 16), lambda i, o=o: (o * _OFF3 + i, 0))
             for o in range(4)]
    return pl.pallas_call(
        _stage3_body,
        grid=(_GRID3,),
        in_specs=zspec + dspec + [
            pl.BlockSpec((1, F), lambda i: (0, 0)),
            pl.BlockSpec((1, F), lambda i: (0, 0)),
            pl.BlockSpec((16, F), lambda i: (0, 0)),
            pl.BlockSpec((16, F), lambda i: (0, 0)),
            pl.BlockSpec((F, HID), lambda i: (0, 0)),
            pl.BlockSpec((1, HID), lambda i: (0, 0)),
            pl.BlockSpec((1, HID), lambda i: (0, 0)),
        ],
        out_specs=pl.BlockSpec((_BLK3, F), lambda i: (i, 0)),
        out_shape=jax.ShapeDtypeStruct((NP, F), jnp.float32),
    )(zcat, zcat, zcat, zcat, dcat, dcat, dcat, dcat,
      bias_0.reshape(1, F), bias_1.reshape(1, F), SLO, SHI,
      sem_W1, sem_b1.reshape(1, HID), sem_W2.reshape(1, HID))


def _attn_mat(attn):
    mask = jnp.repeat(jnp.eye(H, dtype=jnp.float32), D_OUT, axis=0)
    return mask * jnp.tile(attn.T, (H, 1))


def kernel(x, edge_index_0, edge_index_1, W_0, attn_l_0, attn_r_0, bias_0,
           W_1, attn_l_1, attn_r_1, bias_1, sem_W1, sem_b1, sem_W2, sem_b2):
    del sem_b2
    B_0 = jnp.concatenate([_attn_mat(attn_l_0), _attn_mat(attn_r_0)], axis=1)
    B_1 = jnp.concatenate([_attn_mat(attn_l_1), _attn_mat(attn_r_1)], axis=1)
    Wst = jnp.stack([W_0, W_1], axis=0)
    WB16 = jnp.stack([W_0 @ B_0, W_1 @ B_1], axis=0)
    featall4, elrall3 = _stage1(x, Wst, WB16)
    featall = featall4.reshape(4 * N, FH)
    elrall = elrall3.reshape(2 * N, 16)
    srcall = jnp.concatenate([edge_index_0[0], edge_index_1[0]], axis=0)
    dstall = jnp.concatenate([edge_index_0[1], edge_index_1[1]], axis=0)
    sdall = jnp.stack(
        [srcall.reshape(2, NSUB, NCHUNK, CH), dstall.reshape(2, NSUB, NCHUNK, CH)],
        axis=3).reshape(-1)
    zcat, dcat = _stage2(featall, elrall, sdall)
    base = jnp.repeat(jnp.eye(H, dtype=jnp.float32), D_OUT, axis=1)
    zpad = jnp.zeros((12, F), jnp.float32)
    SLO = jnp.concatenate([base[:HH], zpad], axis=0)
    SHI = jnp.concatenate([base[HH:], zpad], axis=0)
    return _stage3(zcat, dcat, bias_0, bias_1, SLO, SHI,
                   sem_W1, sem_b1, sem_W2)[:N]
